# baseline (speedup 1.0000x reference)
import functools

import numpy as np
import jax
import jax.numpy as jnp
from jax import lax
from jax.experimental import pallas as pl
from jax.experimental.pallas import tpu as pltpu
from jax.experimental.pallas import tpu_sc as plsc

DIM = 128
N_NODES = 10000
N_EDGES = 320000
N_RBF = 16
CUTOFF_G = 10.0
ENV_EXP = 5
OUT_DIM = 15

NC = 2
NS = 16
NW = NC * NS
LANES = 16

_P_IDX = np.zeros((DIM,), dtype=np.int32)
for _g in range(DIM // 32):
    for _i in range(16):
        _P_IDX[32 * _g + 2 * _i] = 32 * _g + _i
        _P_IDX[32 * _g + 2 * _i + 1] = 32 * _g + 16 + _i


_EPT = N_EDGES // NW


def _dist2_body(px_h, py_h, pz_h, src_h, dst_h, d2_h,
                px_v, py_v, pz_v, src_v, dst_v, d2_v):
    c = lax.axis_index("c")
    s = lax.axis_index("s")
    wid = c * NS + s
    base = wid * _EPT
    pltpu.sync_copy(px_h, px_v)
    pltpu.sync_copy(py_h, py_v)
    pltpu.sync_copy(pz_h, pz_v)
    pltpu.sync_copy(src_h.at[pl.ds(base, _EPT)], src_v)
    pltpu.sync_copy(dst_h.at[pl.ds(base, _EPT)], dst_v)

    def step(i, _):
        sl = pl.ds(i * LANES, LANES)
        si = src_v[sl]
        di = dst_v[sl]
        dx = plsc.load_gather(px_v, [di]) - plsc.load_gather(px_v, [si])
        dy = plsc.load_gather(py_v, [di]) - plsc.load_gather(py_v, [si])
        dz = plsc.load_gather(pz_v, [di]) - plsc.load_gather(pz_v, [si])
        d2_v[sl] = dx * dx + dy * dy + dz * dz
        return 0

    lax.fori_loop(0, _EPT // LANES, step, 0)
    pltpu.sync_copy(d2_v, d2_h.at[pl.ds(base, _EPT)])


def _dist2_call(px, py, pz, src, dst):
    mesh = plsc.VectorSubcoreMesh(core_axis_name="c", subcore_axis_name="s",
                                  num_cores=NC, num_subcores=NS)
    return pl.kernel(
        _dist2_body,
        out_type=jax.ShapeDtypeStruct((N_EDGES,), jnp.float32),
        mesh=mesh,
        compiler_params=pltpu.CompilerParams(needs_layout_passes=False),
        scratch_types=[
            pltpu.VMEM((N_NODES,), jnp.float32),
            pltpu.VMEM((N_NODES,), jnp.float32),
            pltpu.VMEM((N_NODES,), jnp.float32),
            pltpu.VMEM((_EPT,), jnp.int32),
            pltpu.VMEM((_EPT,), jnp.int32),
            pltpu.VMEM((_EPT,), jnp.float32),
        ],
    )(px, py, pz, src, dst)


_K = 80
_NCHUNK = _EPT // _K


def _mul_chunk(rows_v, ew_v, prod_v):
    def mul_row(r, _):
        for jj in range(DIM // 32):
            sl16 = pl.ds(16 * jj, 16)
            xw = plsc.bitcast(rows_v[r, sl16], jnp.bfloat16)
            ew = plsc.bitcast(ew_v[r, sl16], jnp.bfloat16)
            xa, xb = plsc.unpack(xw, format=plsc.PackFormat.INTERLEAVED)
            ea, eb = plsc.unpack(ew, format=plsc.PackFormat.INTERLEAVED)
            prod_v[r, pl.ds(32 * jj, 16)] = xa * ea
            prod_v[r, pl.ds(32 * jj + 16, 16)] = xb * eb
        return 0

    lax.fori_loop(0, _K, mul_row, 0)


def _gms_body(x_h, ew_h, src_h, dst3_h, zero_h, out_h,
              src_v, dst2d, rows_a, rows_b, ew_a, ew_b, prod_v, acc,
              gsem_a, gsem_b, ssem):
    c = lax.axis_index("c")
    s = lax.axis_index("s")
    wid = c * NS + s
    ebase = wid * _EPT

    pltpu.sync_copy(src_h.at[pl.ds(ebase, _EPT)], src_v)
    pltpu.sync_copy(dst3_h.at[wid], dst2d)

    def _zero():
        def zc(i, _):
            ch = s + i * NS

            @pl.when(ch < N_NODES // 200)
            def _():
                pltpu.sync_copy(zero_h.at[pl.ds(ch * 200, 200)],
                                acc.at[pl.ds(ch * 200, 200)])
            return 0

        lax.fori_loop(0, (N_NODES // 200 + NS - 1) // NS, zc, 0)

    if True:
        _zero()
        plsc.subcore_barrier()

        def issue(i, rows_v, ew_v, sem):
            eb = ebase + i * _K
            pltpu.async_copy(x_h.at[src_v.at[pl.ds(i * _K, _K)]], rows_v, sem)
            pltpu.async_copy(ew_h.at[pl.ds(eb, _K)], ew_v, sem)

        def wait(rows_v, ew_v, sem):
            pltpu.make_async_copy(x_h.at[src_v.at[pl.ds(0, _K)]], rows_v, sem).wait()
            pltpu.make_async_copy(ew_h.at[pl.ds(0, _K)], ew_v, sem).wait()

        def do_chunk(i, rows_v, ew_v, sem, rows_n, ew_n, sem_n):
            wait(rows_v, ew_v, sem)

            @pl.when(i + 1 < _NCHUNK)
            def _():
                issue(i + 1, rows_n, ew_n, sem_n)

            @pl.when(i > 0)
            def _():
                pltpu.make_async_copy(prod_v, acc.at[dst2d.at[0]], ssem).wait()
            _mul_chunk(rows_v, ew_v, prod_v)
            pltpu.async_copy(prod_v, acc.at[dst2d.at[i]], ssem, add=True)

        issue(0, rows_a, ew_a, gsem_a)

        def body(i, _):
            @pl.when(i % 2 == 0)
            def _():
                do_chunk(i, rows_a, ew_a, gsem_a, rows_b, ew_b, gsem_b)

            @pl.when(i % 2 == 1)
            def _():
                do_chunk(i, rows_b, ew_b, gsem_b, rows_a, ew_a, gsem_a)
            return 0

        lax.fori_loop(0, _NCHUNK, body, 0)
        pltpu.make_async_copy(prod_v, acc.at[dst2d.at[0]], ssem).wait()
        plsc.subcore_barrier()

        def rb(i, _):
            ch = s + i * NS

            @pl.when(ch < N_NODES // 200)
            def _():
                pltpu.sync_copy(acc.at[pl.ds(ch * 200, 200)],
                                out_h.at[c, pl.ds(ch * 200, 200)])
            return 0

        lax.fori_loop(0, (N_NODES // 200 + NS - 1) // NS, rb, 0)


def _gather_mul_scatter(xb, ewb, src, dst3, zeros):
    mesh = plsc.VectorSubcoreMesh(core_axis_name="c", subcore_axis_name="s",
                                  num_cores=NC, num_subcores=NS)
    return pl.kernel(
        _gms_body,
        out_type=jax.ShapeDtypeStruct((NC, N_NODES, DIM), jnp.float32),
        mesh=mesh,
        compiler_params=pltpu.CompilerParams(needs_layout_passes=False,
                                             use_tc_tiling_on_sc=False),
        scratch_types=[
            pltpu.VMEM((_EPT,), jnp.int32),
            pltpu.VMEM((_NCHUNK, _K), jnp.int32),
            pltpu.VMEM((_K, DIM // 2), jnp.int32),
            pltpu.VMEM((_K, DIM // 2), jnp.int32),
            pltpu.VMEM((_K, DIM // 2), jnp.int32),
            pltpu.VMEM((_K, DIM // 2), jnp.int32),
            pltpu.VMEM((_K, DIM), jnp.float32),
            pltpu.VMEM_SHARED((N_NODES, DIM), jnp.float32),
            pltpu.SemaphoreType.DMA,
            pltpu.SemaphoreType.DMA,
            pltpu.SemaphoreType.DMA,
        ],
    )(xb, ewb, src, dst3, zeros)


_BN = 2000


def _init_tc_body(pos_ref, wp_ref, x_ref, xb_ref):
    p = pos_ref[...]
    wp = wp_ref[...]
    acc = p[:, 0:1] * wp[0:1, :]
    acc += p[:, 1:2] * wp[1:2, :]
    acc += p[:, 2:3] * wp[2:3, :]
    xp = jnp.maximum(acc, 0.0)
    x_ref[...] = xp
    xb_ref[...] = xp.astype(jnp.bfloat16)


def _init_tc(pos, W_init_p):
    return pl.pallas_call(
        _init_tc_body,
        grid=(N_NODES // _BN,),
        in_specs=[
            pl.BlockSpec((_BN, 3), lambda i: (i, 0)),
            pl.BlockSpec((3, DIM), lambda i: (0, 0)),
        ],
        out_specs=[
            pl.BlockSpec((_BN, DIM), lambda i: (i, 0)),
            pl.BlockSpec((_BN, DIM), lambda i: (i, 0)),
        ],
        out_shape=[
            jax.ShapeDtypeStruct((N_NODES, DIM), jnp.float32),
            jax.ShapeDtypeStruct((N_NODES, DIM), jnp.bfloat16),
        ],
    )(pos, W_init_p)


_BE = 2560


def _ew_tc_body(d2_ref, freqs_ref, wrbf_ref, ew_ref):
    d2 = d2_ref[...]
    dist = jnp.sqrt(d2 + 1e-12)
    d = dist * (1.0 / CUTOFF_G)
    p = ENV_EXP + 1
    a = -(p + 1) * (p + 2) / 2.0
    b = p * (p + 2)
    cc = -p * (p + 1) / 2.0
    d_safe = jnp.maximum(d, 1e-6)
    d4 = (d_safe * d_safe) * (d_safe * d_safe)
    d5 = d4 * d_safe
    env = 1.0 / d_safe + a * d5 + b * d5 * d_safe + cc * d5 * d_safe * d_safe
    env = jnp.where(d < 1.0, env, 0.0)
    rbf = env * jnp.sin(d * freqs_ref[...])
    ew = jnp.maximum(
        jnp.dot(rbf, wrbf_ref[...], preferred_element_type=jnp.float32), 0.0)
    ew_ref[...] = ew.astype(jnp.bfloat16)


def _ew_tc(d2, freqs_row, W_rbf_p):
    return pl.pallas_call(
        _ew_tc_body,
        grid=(N_EDGES // _BE,),
        in_specs=[
            pl.BlockSpec((_BE, 1), lambda i: (i, 0)),
            pl.BlockSpec((1, N_RBF), lambda i: (0, 0)),
            pl.BlockSpec((N_RBF, DIM), lambda i: (0, 0)),
        ],
        out_specs=pl.BlockSpec((_BE, DIM), lambda i: (i, 0)),
        out_shape=jax.ShapeDtypeStruct((N_EDGES, DIM), jnp.bfloat16),
    )(d2, freqs_row, W_rbf_p)


def _upd_tc_body(x_ref, g_ref, wm_ref, wup_ref, xo_ref, xb_ref):
    gsum = g_ref[0] + g_ref[1]
    aggr = jnp.dot(gsum, wm_ref[...], preferred_element_type=jnp.float32)
    h = jnp.dot(aggr, wup_ref[...], preferred_element_type=jnp.float32)
    xn = jnp.maximum(x_ref[...] + h, 0.0)
    xo_ref[...] = xn
    xb_ref[...] = xn.astype(jnp.bfloat16)


def _upd_tc(x, g, Wm, Wup):
    return pl.pallas_call(
        _upd_tc_body,
        grid=(N_NODES // _BN,),
        in_specs=[
            pl.BlockSpec((_BN, DIM), lambda i: (i, 0)),
            pl.BlockSpec((NC, _BN, DIM), lambda i: (0, i, 0)),
            pl.BlockSpec((DIM, DIM), lambda i: (0, 0)),
            pl.BlockSpec((DIM, DIM), lambda i: (0, 0)),
        ],
        out_specs=[
            pl.BlockSpec((_BN, DIM), lambda i: (i, 0)),
            pl.BlockSpec((_BN, DIM), lambda i: (i, 0)),
        ],
        out_shape=[
            jax.ShapeDtypeStruct((N_NODES, DIM), jnp.float32),
            jax.ShapeDtypeStruct((N_NODES, DIM), jnp.bfloat16),
        ],
    )(x, g, Wm, Wup)


def _final_tc_body(x_ref, g_ref, wm_ref, wup_ref, wo_ref, out_ref):
    gsum = g_ref[0] + g_ref[1]
    aggr = jnp.dot(gsum, wm_ref[...], preferred_element_type=jnp.float32)
    h = jnp.dot(aggr, wup_ref[...], preferred_element_type=jnp.float32)
    x2 = jnp.maximum(x_ref[...] + h, 0.0)
    out_ref[...] = jnp.dot(x2, wo_ref[...], preferred_element_type=jnp.float32)


def _final_tc(x, g, Wm, Wup, Wo_pad):
    return pl.pallas_call(
        _final_tc_body,
        grid=(N_NODES // _BN,),
        in_specs=[
            pl.BlockSpec((_BN, DIM), lambda i: (i, 0)),
            pl.BlockSpec((NC, _BN, DIM), lambda i: (0, i, 0)),
            pl.BlockSpec((DIM, DIM), lambda i: (0, 0)),
            pl.BlockSpec((DIM, DIM), lambda i: (0, 0)),
            pl.BlockSpec((DIM, DIM), lambda i: (0, 0)),
        ],
        out_specs=pl.BlockSpec((_BN, DIM), lambda i: (i, 0)),
        out_shape=jax.ShapeDtypeStruct((N_NODES, DIM), jnp.float32),
    )(x, g, Wm, Wup, Wo_pad)


@jax.jit
def _run(pos, edge_index, W_init, freqs, W_rbf, W_msg, W_upd, W_out):
    perm = jnp.asarray(_P_IDX)
    src = edge_index[0]
    dst = edge_index[1]
    dst3 = dst.reshape(NW, _NCHUNK, _K)
    px = pos[:, 0]
    py = pos[:, 1]
    pz = pos[:, 2]
    zeros = jnp.zeros((N_NODES, DIM), jnp.float32)

    d2 = _dist2_call(px, py, pz, src, dst)
    ew = _ew_tc(d2.reshape(N_EDGES, 1), freqs.reshape(1, N_RBF),
                W_rbf[:, perm])
    x, xb = _init_tc(pos, W_init[:, perm])

    ew32 = lax.bitcast_convert_type(ew.reshape(N_EDGES, DIM // 2, 2),
                                    jnp.int32)

    def as32(a):
        return lax.bitcast_convert_type(a.reshape(N_NODES, DIM // 2, 2),
                                        jnp.int32)

    g = _gather_mul_scatter(as32(xb), ew32, src, dst3, zeros)
    x, xb = _upd_tc(x, g, W_msg[0], W_upd[0][:, perm])

    g = _gather_mul_scatter(as32(xb), ew32, src, dst3, zeros)
    Wo_pad = jnp.pad(W_out[perm, :], ((0, 0), (0, DIM - OUT_DIM)))
    out = _final_tc(x, g, W_msg[1], W_upd[1][:, perm], Wo_pad)
    return out[:, :OUT_DIM]


def kernel(pos, edge_index, W_init, freqs, W_rbf, W_msg, W_upd, W_out):
    return _run(pos, edge_index, W_init, freqs, W_rbf, W_msg, W_upd, W_out)


# speedup vs baseline: 1.1820x; 1.1820x over previous
import functools

import numpy as np
import jax
import jax.numpy as jnp
from jax import lax
from jax.experimental import pallas as pl
from jax.experimental.pallas import tpu as pltpu
from jax.experimental.pallas import tpu_sc as plsc

DIM = 128
N_NODES = 10000
N_EDGES = 320000
N_RBF = 16
CUTOFF_G = 10.0
ENV_EXP = 5
OUT_DIM = 15

NC = 2
NS = 16
NW = NC * NS
LANES = 16

_P_IDX = np.zeros((DIM,), dtype=np.int32)
for _g in range(DIM // 32):
    for _i in range(16):
        _P_IDX[32 * _g + 2 * _i] = 32 * _g + _i
        _P_IDX[32 * _g + 2 * _i + 1] = 32 * _g + 16 + _i


_EPT = N_EDGES // NW


def _dist2_body(px_h, py_h, pz_h, src_h, dst_h, d2_h,
                px_v, py_v, pz_v, src_v, dst_v, d2_v):
    c = lax.axis_index("c")
    s = lax.axis_index("s")
    wid = c * NS + s
    base = wid * _EPT
    pltpu.sync_copy(px_h, px_v)
    pltpu.sync_copy(py_h, py_v)
    pltpu.sync_copy(pz_h, pz_v)
    pltpu.sync_copy(src_h.at[pl.ds(base, _EPT)], src_v)
    pltpu.sync_copy(dst_h.at[pl.ds(base, _EPT)], dst_v)

    def step(i, _):
        sl = pl.ds(i * LANES, LANES)
        si = src_v[sl]
        di = dst_v[sl]
        dx = plsc.load_gather(px_v, [di]) - plsc.load_gather(px_v, [si])
        dy = plsc.load_gather(py_v, [di]) - plsc.load_gather(py_v, [si])
        dz = plsc.load_gather(pz_v, [di]) - plsc.load_gather(pz_v, [si])
        d2_v[sl] = dx * dx + dy * dy + dz * dz
        return 0

    lax.fori_loop(0, _EPT // LANES, step, 0)
    pltpu.sync_copy(d2_v, d2_h.at[pl.ds(base, _EPT)])


def _dist2_call(px, py, pz, src, dst):
    mesh = plsc.VectorSubcoreMesh(core_axis_name="c", subcore_axis_name="s",
                                  num_cores=NC, num_subcores=NS)
    return pl.kernel(
        _dist2_body,
        out_type=jax.ShapeDtypeStruct((N_EDGES,), jnp.float32),
        mesh=mesh,
        compiler_params=pltpu.CompilerParams(needs_layout_passes=False),
        scratch_types=[
            pltpu.VMEM((N_NODES,), jnp.float32),
            pltpu.VMEM((N_NODES,), jnp.float32),
            pltpu.VMEM((N_NODES,), jnp.float32),
            pltpu.VMEM((_EPT,), jnp.int32),
            pltpu.VMEM((_EPT,), jnp.int32),
            pltpu.VMEM((_EPT,), jnp.float32),
        ],
    )(px, py, pz, src, dst)


_K = 80
_NCHUNK = _EPT // _K
_KQ = _K // 16


def _mul_chunk(prod_v, ew_v):
    def mul_row(r, _):
        for jj in range(DIM // 32):
            ew = plsc.bitcast(ew_v[r, pl.ds(16 * jj, 16)], jnp.bfloat16)
            ea, eb = plsc.unpack(ew, format=plsc.PackFormat.INTERLEAVED)
            sa = pl.ds(32 * jj, 16)
            sb = pl.ds(32 * jj + 16, 16)
            prod_v[r, sa] = prod_v[r, sa] * ea
            prod_v[r, sb] = prod_v[r, sb] * eb
        return 0

    lax.fori_loop(0, _K, mul_row, 0)


def _gms_body(x_h, ew3_h, src_h, dst_h, zero_h, out_h,
              src_v, pr_a, pr_b, ew_a, ew_b, dst_a, dst_b, acc,
              gsem_a, gsem_b, ssem):
    c = lax.axis_index("c")
    s = lax.axis_index("s")
    wid = c * NS + s
    ebase = wid * _EPT

    _SBLK = 2000

    def load_src(blk):
        pltpu.sync_copy(src_h.at[pl.ds(ebase + blk * _SBLK, _SBLK)], src_v)

    load_src(0)

    def zc(i, _):
        ch = s + i * NS

        @pl.when(ch < N_NODES // 200)
        def _():
            pltpu.sync_copy(zero_h.at[pl.ds(ch * 200, 200)],
                            acc.at[pl.ds(ch * 200, 200)])
        return 0

    lax.fori_loop(0, (N_NODES // 200 + NS - 1) // NS, zc, 0)
    plsc.subcore_barrier()

    def issue(i, pr_v, ew_v, dst_v, sem):
        eb = ebase + i * _K
        pltpu.async_copy(dst_h.at[pl.ds(eb, _K)], dst_v, sem)
        pltpu.async_copy(ew3_h.at[pl.ds(eb, _K)], ew_v, sem)
        off = (i % (_SBLK // _K)) * _K
        pltpu.async_copy(x_h.at[src_v.at[pl.ds(off, _K)]], pr_v, sem)

    def wait_in(pr_v, ew_v, dst_v, sem):
        pltpu.make_async_copy(dst_h.at[pl.ds(0, _K)], dst_v, sem).wait()
        pltpu.make_async_copy(ew3_h.at[pl.ds(0, _K)], ew_v, sem).wait()
        pltpu.make_async_copy(x_h.at[src_v.at[pl.ds(0, _K)]], pr_v, sem).wait()

    def drain_scatter():
        pltpu.make_async_copy(pr_a, acc.at[dst_a], ssem).wait()

    def do_chunk(i, pr_v, ew_v, dst_v, sem, pr_n, ew_n, dst_n, sem_n):
        wait_in(pr_v, ew_v, dst_v, sem)

        @pl.when(i > 0)
        def _():
            drain_scatter()

        @pl.when((i + 1) % (_SBLK // _K) == 0)
        def _():
            load_src((i + 1) // (_SBLK // _K))

        @pl.when(i + 1 < _NCHUNK)
        def _():
            issue(i + 1, pr_n, ew_n, dst_n, sem_n)

        _mul_chunk(pr_v, ew_v)
        pltpu.async_copy(pr_v, acc.at[dst_v], ssem, add=True)

    issue(0, pr_a, ew_a, dst_a, gsem_a)

    def body(i, _):
        @pl.when(i % 2 == 0)
        def _():
            do_chunk(i, pr_a, ew_a, dst_a, gsem_a, pr_b, ew_b, dst_b, gsem_b)

        @pl.when(i % 2 == 1)
        def _():
            do_chunk(i, pr_b, ew_b, dst_b, gsem_b, pr_a, ew_a, dst_a, gsem_a)
        return 0

    lax.fori_loop(0, _NCHUNK, body, 0)
    drain_scatter()
    plsc.subcore_barrier()

    def rb(i, _):
        ch = s + i * NS

        @pl.when(ch < N_NODES // 200)
        def _():
            pltpu.sync_copy(acc.at[pl.ds(ch * 200, 200)],
                            out_h.at[c, pl.ds(ch * 200, 200)])
        return 0

    lax.fori_loop(0, (N_NODES // 200 + NS - 1) // NS, rb, 0)


def _gather_mul_scatter(x, ew3, src, dst, zeros):
    mesh = plsc.VectorSubcoreMesh(core_axis_name="c", subcore_axis_name="s",
                                  num_cores=NC, num_subcores=NS)
    return pl.kernel(
        _gms_body,
        out_type=jax.ShapeDtypeStruct((NC, N_NODES, DIM), jnp.float32),
        mesh=mesh,
        compiler_params=pltpu.CompilerParams(needs_layout_passes=False),
        scratch_types=[
            pltpu.VMEM((2000,), jnp.int32),
            pltpu.VMEM((_K, DIM), jnp.float32),
            pltpu.VMEM((_K, DIM), jnp.float32),
            pltpu.VMEM((_K, DIM // 2), jnp.int32),
            pltpu.VMEM((_K, DIM // 2), jnp.int32),
            pltpu.VMEM((_K,), jnp.int32),
            pltpu.VMEM((_K,), jnp.int32),
            pltpu.VMEM_SHARED((N_NODES, DIM), jnp.float32),
            pltpu.SemaphoreType.DMA,
            pltpu.SemaphoreType.DMA,
            pltpu.SemaphoreType.DMA,
        ],
    )(x, ew3, src, dst, zeros)


_BN = 2000


def _init_tc_body(pos_ref, w_ref, x_ref):
    p = pos_ref[...]
    w = w_ref[...]
    acc = p[:, 0:1] * w[0:1, :]
    acc += p[:, 1:2] * w[1:2, :]
    acc += p[:, 2:3] * w[2:3, :]
    x_ref[...] = jnp.maximum(acc, 0.0)


def _init_tc(pos, W_init):
    return pl.pallas_call(
        _init_tc_body,
        grid=(N_NODES // _BN,),
        in_specs=[
            pl.BlockSpec((_BN, 3), lambda i: (i, 0)),
            pl.BlockSpec((3, DIM), lambda i: (0, 0)),
        ],
        out_specs=pl.BlockSpec((_BN, DIM), lambda i: (i, 0)),
        out_shape=jax.ShapeDtypeStruct((N_NODES, DIM), jnp.float32),
    )(pos, W_init)


_BE = 2560


def _ew_tc_body(d2_ref, freqs_ref, wrbf_ref, ew_ref):
    d2 = d2_ref[...]
    dist = jnp.sqrt(d2 + 1e-12)
    d = dist * (1.0 / CUTOFF_G)
    p = ENV_EXP + 1
    a = -(p + 1) * (p + 2) / 2.0
    b = p * (p + 2)
    cc = -p * (p + 1) / 2.0
    d_safe = jnp.maximum(d, 1e-6)
    d4 = (d_safe * d_safe) * (d_safe * d_safe)
    d5 = d4 * d_safe
    env = 1.0 / d_safe + a * d5 + b * d5 * d_safe + cc * d5 * d_safe * d_safe
    env = jnp.where(d < 1.0, env, 0.0)
    rbf = env * jnp.sin(d * freqs_ref[...])
    ew = jnp.maximum(
        jnp.dot(rbf, wrbf_ref[...], preferred_element_type=jnp.float32), 0.0)
    ew_ref[...] = ew.astype(jnp.bfloat16)


def _ew_tc(d2, freqs_row, W_rbf_p):
    return pl.pallas_call(
        _ew_tc_body,
        grid=(N_EDGES // _BE,),
        in_specs=[
            pl.BlockSpec((_BE, 1), lambda i: (i, 0)),
            pl.BlockSpec((1, N_RBF), lambda i: (0, 0)),
            pl.BlockSpec((N_RBF, DIM), lambda i: (0, 0)),
        ],
        out_specs=pl.BlockSpec((_BE, DIM), lambda i: (i, 0)),
        out_shape=jax.ShapeDtypeStruct((N_EDGES, DIM), jnp.bfloat16),
    )(d2, freqs_row, W_rbf_p)


def _upd_tc_body(x_ref, g_ref, wm_ref, wu_ref, xo_ref):
    gsum = g_ref[0] + g_ref[1]
    aggr = jnp.dot(gsum, wm_ref[...], preferred_element_type=jnp.float32)
    h = jnp.dot(aggr, wu_ref[...], preferred_element_type=jnp.float32)
    xo_ref[...] = jnp.maximum(x_ref[...] + h, 0.0)


def _upd_tc(x, g, Wm, Wu):
    return pl.pallas_call(
        _upd_tc_body,
        grid=(N_NODES // _BN,),
        in_specs=[
            pl.BlockSpec((_BN, DIM), lambda i: (i, 0)),
            pl.BlockSpec((NC, _BN, DIM), lambda i: (0, i, 0)),
            pl.BlockSpec((DIM, DIM), lambda i: (0, 0)),
            pl.BlockSpec((DIM, DIM), lambda i: (0, 0)),
        ],
        out_specs=pl.BlockSpec((_BN, DIM), lambda i: (i, 0)),
        out_shape=jax.ShapeDtypeStruct((N_NODES, DIM), jnp.float32),
    )(x, g, Wm, Wu)


def _final_tc_body(x_ref, g_ref, wm_ref, wup_ref, wo_ref, out_ref):
    gsum = g_ref[0] + g_ref[1]
    aggr = jnp.dot(gsum, wm_ref[...], preferred_element_type=jnp.float32)
    h = jnp.dot(aggr, wup_ref[...], preferred_element_type=jnp.float32)
    x2 = jnp.maximum(x_ref[...] + h, 0.0)
    out_ref[...] = jnp.dot(x2, wo_ref[...], preferred_element_type=jnp.float32)


def _final_tc(x, g, Wm, Wup, Wo_pad):
    return pl.pallas_call(
        _final_tc_body,
        grid=(N_NODES // _BN,),
        in_specs=[
            pl.BlockSpec((_BN, DIM), lambda i: (i, 0)),
            pl.BlockSpec((NC, _BN, DIM), lambda i: (0, i, 0)),
            pl.BlockSpec((DIM, DIM), lambda i: (0, 0)),
            pl.BlockSpec((DIM, DIM), lambda i: (0, 0)),
            pl.BlockSpec((DIM, DIM), lambda i: (0, 0)),
        ],
        out_specs=pl.BlockSpec((_BN, DIM), lambda i: (i, 0)),
        out_shape=jax.ShapeDtypeStruct((N_NODES, DIM), jnp.float32),
    )(x, g, Wm, Wup, Wo_pad)


@jax.jit
def _run(pos, edge_index, W_init, freqs, W_rbf, W_msg, W_upd, W_out):
    perm = jnp.asarray(_P_IDX)
    src = edge_index[0]
    dst = edge_index[1]
    px = pos[:, 0]
    py = pos[:, 1]
    pz = pos[:, 2]
    zeros = jnp.zeros((N_NODES, DIM), jnp.float32)

    d2 = _dist2_call(px, py, pz, src, dst)
    ew = _ew_tc(d2.reshape(N_EDGES, 1), freqs.reshape(1, N_RBF),
                W_rbf[:, perm])
    ew3 = lax.bitcast_convert_type(ew.reshape(N_EDGES, DIM // 2, 2),
                                   jnp.int32)
    x = _init_tc(pos, W_init)

    g = _gather_mul_scatter(x, ew3, src, dst, zeros)
    x = _upd_tc(x, g, W_msg[0], W_upd[0])

    g = _gather_mul_scatter(x, ew3, src, dst, zeros)
    Wo_pad = jnp.pad(W_out, ((0, 0), (0, DIM - OUT_DIM)))
    out = _final_tc(x, g, W_msg[1], W_upd[1], Wo_pad)
    return out[:, :OUT_DIM]


def kernel(pos, edge_index, W_init, freqs, W_rbf, W_msg, W_upd, W_out):
    return _run(pos, edge_index, W_init, freqs, W_rbf, W_msg, W_upd, W_out)


# speedup vs baseline: 1.9951x; 1.6880x over previous
import functools

import numpy as np
import jax
import jax.numpy as jnp
from jax import lax
from jax.experimental import pallas as pl
from jax.experimental.pallas import tpu as pltpu
from jax.experimental.pallas import tpu_sc as plsc

DIM = 128
N_NODES = 10000
N_EDGES = 320000
N_RBF = 16
CUTOFF_G = 10.0
ENV_EXP = 5
OUT_DIM = 15

NC = 2
NS = 16
NW = NC * NS
LANES = 16


_EPT = N_EDGES // NW


def _dist2_body(px_h, py_h, pz_h, src_h, dst_h, d2_h,
                px_v, py_v, pz_v, src_v, dst_v, d2_v):
    c = lax.axis_index("c")
    s = lax.axis_index("s")
    wid = c * NS + s
    base = wid * _EPT
    pltpu.sync_copy(px_h, px_v)
    pltpu.sync_copy(py_h, py_v)
    pltpu.sync_copy(pz_h, pz_v)
    pltpu.sync_copy(src_h.at[pl.ds(base, _EPT)], src_v)
    pltpu.sync_copy(dst_h.at[pl.ds(base, _EPT)], dst_v)

    def step(i, _):
        sl = pl.ds(i * LANES, LANES)
        si = src_v[sl]
        di = dst_v[sl]
        dx = plsc.load_gather(px_v, [di]) - plsc.load_gather(px_v, [si])
        dy = plsc.load_gather(py_v, [di]) - plsc.load_gather(py_v, [si])
        dz = plsc.load_gather(pz_v, [di]) - plsc.load_gather(pz_v, [si])
        d2_v[sl] = dx * dx + dy * dy + dz * dz
        return 0

    lax.fori_loop(0, _EPT // LANES, step, 0)
    pltpu.sync_copy(d2_v, d2_h.at[pl.ds(base, _EPT)])


def _dist2_call(px, py, pz, src, dst):
    mesh = plsc.VectorSubcoreMesh(core_axis_name="c", subcore_axis_name="s",
                                  num_cores=NC, num_subcores=NS)
    return pl.kernel(
        _dist2_body,
        out_type=jax.ShapeDtypeStruct((N_EDGES,), jnp.float32),
        mesh=mesh,
        compiler_params=pltpu.CompilerParams(needs_layout_passes=False),
        scratch_types=[
            pltpu.VMEM((N_NODES,), jnp.float32),
            pltpu.VMEM((N_NODES,), jnp.float32),
            pltpu.VMEM((N_NODES,), jnp.float32),
            pltpu.VMEM((_EPT,), jnp.int32),
            pltpu.VMEM((_EPT,), jnp.int32),
            pltpu.VMEM((_EPT,), jnp.float32),
        ],
    )(px, py, pz, src, dst)


_K = 80
_NCHUNK = _EPT // _K


def _mul_chunk(prod_v, ew_v):
    def mul_row(r, _):
        for jj in range(DIM // LANES):
            sl = pl.ds(LANES * jj, LANES)
            prod_v[r, sl] = prod_v[r, sl] * ew_v[r, sl]
        return 0

    lax.fori_loop(0, _K, mul_row, 0)


def _gms_body(x_h, ew3_h, src_h, dst_h, out_h,
              src_v, pr_a, pr_b, ew_a, ew_b, dst_a, dst_b, zbuf, acc,
              gsem_a, gsem_b, ssem):
    c = lax.axis_index("c")
    s = lax.axis_index("s")
    wid = c * NS + s
    ebase = wid * _EPT

    _SBLK = 2000

    def load_src(blk):
        pltpu.sync_copy(src_h.at[pl.ds(ebase + blk * _SBLK, _SBLK)], src_v)

    load_src(0)

    for zr in range(16):
        for jj in range(DIM // LANES):
            zbuf[zr, pl.ds(jj * LANES, LANES)] = jnp.zeros((LANES,),
                                                           jnp.float32)

    def zc(i, _):
        ch = s + i * NS

        @pl.when(ch < N_NODES // 16)
        def _():
            pltpu.sync_copy(zbuf, acc.at[pl.ds(ch * 16, 16)])
        return 0

    lax.fori_loop(0, (N_NODES // 16 + NS - 1) // NS, zc, 0)
    plsc.subcore_barrier()

    def issue(i, pr_v, ew_v, dst_v, sem):
        eb = ebase + i * _K
        pltpu.async_copy(dst_h.at[pl.ds(eb, _K)], dst_v, sem)
        pltpu.async_copy(ew3_h.at[pl.ds(eb, _K)], ew_v, sem)
        off = (i % (_SBLK // _K)) * _K
        pltpu.async_copy(x_h.at[src_v.at[pl.ds(off, _K)]], pr_v, sem)

    def wait_in(pr_v, ew_v, dst_v, sem):
        pltpu.make_async_copy(dst_h.at[pl.ds(0, _K)], dst_v, sem).wait()
        pltpu.make_async_copy(ew3_h.at[pl.ds(0, _K)], ew_v, sem).wait()
        pltpu.make_async_copy(x_h.at[src_v.at[pl.ds(0, _K)]], pr_v, sem).wait()

    def drain_scatter():
        pltpu.make_async_copy(pr_a, acc.at[dst_a], ssem).wait()

    def do_chunk(i, pr_v, ew_v, dst_v, sem, pr_n, ew_n, dst_n, sem_n):
        wait_in(pr_v, ew_v, dst_v, sem)

        @pl.when(i > 0)
        def _():
            drain_scatter()

        @pl.when((i + 1) % (_SBLK // _K) == 0)
        def _():
            load_src((i + 1) // (_SBLK // _K))

        @pl.when(i + 1 < _NCHUNK)
        def _():
            issue(i + 1, pr_n, ew_n, dst_n, sem_n)

        _mul_chunk(pr_v, ew_v)
        pltpu.async_copy(pr_v, acc.at[dst_v], ssem, add=True)

    issue(0, pr_a, ew_a, dst_a, gsem_a)

    def body(i, _):
        @pl.when(i % 2 == 0)
        def _():
            do_chunk(i, pr_a, ew_a, dst_a, gsem_a, pr_b, ew_b, dst_b, gsem_b)

        @pl.when(i % 2 == 1)
        def _():
            do_chunk(i, pr_b, ew_b, dst_b, gsem_b, pr_a, ew_a, dst_a, gsem_a)
        return 0

    lax.fori_loop(0, _NCHUNK, body, 0)
    drain_scatter()
    plsc.subcore_barrier()

    def rb(i, _):
        ch = s + i * NS

        @pl.when(ch < N_NODES // 200)
        def _():
            pltpu.sync_copy(acc.at[pl.ds(ch * 200, 200)],
                            out_h.at[c, pl.ds(ch * 200, 200)])
        return 0

    lax.fori_loop(0, (N_NODES // 200 + NS - 1) // NS, rb, 0)


def _gather_mul_scatter(x, ew, src, dst):
    mesh = plsc.VectorSubcoreMesh(core_axis_name="c", subcore_axis_name="s",
                                  num_cores=NC, num_subcores=NS)
    return pl.kernel(
        _gms_body,
        out_type=jax.ShapeDtypeStruct((NC, N_NODES, DIM), jnp.float32),
        mesh=mesh,
        compiler_params=pltpu.CompilerParams(needs_layout_passes=False),
        scratch_types=[
            pltpu.VMEM((2000,), jnp.int32),
            pltpu.VMEM((_K, DIM), jnp.float32),
            pltpu.VMEM((_K, DIM), jnp.float32),
            pltpu.VMEM((_K, DIM), jnp.float32),
            pltpu.VMEM((_K, DIM), jnp.float32),
            pltpu.VMEM((_K,), jnp.int32),
            pltpu.VMEM((_K,), jnp.int32),
            pltpu.VMEM((16, DIM), jnp.float32),
            pltpu.VMEM_SHARED((N_NODES, DIM), jnp.float32),
            pltpu.SemaphoreType.DMA,
            pltpu.SemaphoreType.DMA,
            pltpu.SemaphoreType.DMA,
        ],
    )(x, ew, src, dst)


_BN = 2000


def _init_tc_body(pos_ref, w_ref, x_ref):
    p = pos_ref[...]
    w = w_ref[...]
    acc = p[:, 0:1] * w[0:1, :]
    acc += p[:, 1:2] * w[1:2, :]
    acc += p[:, 2:3] * w[2:3, :]
    x_ref[...] = jnp.maximum(acc, 0.0)


def _init_tc(pos, W_init):
    return pl.pallas_call(
        _init_tc_body,
        grid=(N_NODES // _BN,),
        in_specs=[
            pl.BlockSpec((_BN, 3), lambda i: (i, 0)),
            pl.BlockSpec((3, DIM), lambda i: (0, 0)),
        ],
        out_specs=pl.BlockSpec((_BN, DIM), lambda i: (i, 0)),
        out_shape=jax.ShapeDtypeStruct((N_NODES, DIM), jnp.float32),
    )(pos, W_init)


_BE = 2560


def _ew_tc_body(d2_ref, freqs_ref, wrbf_ref, ew_ref):
    d2 = d2_ref[...]
    dist = jnp.sqrt(d2 + 1e-12)
    d = dist * (1.0 / CUTOFF_G)
    p = ENV_EXP + 1
    a = -(p + 1) * (p + 2) / 2.0
    b = p * (p + 2)
    cc = -p * (p + 1) / 2.0
    d_safe = jnp.maximum(d, 1e-6)
    d4 = (d_safe * d_safe) * (d_safe * d_safe)
    d5 = d4 * d_safe
    env = 1.0 / d_safe + a * d5 + b * d5 * d_safe + cc * d5 * d_safe * d_safe
    env = jnp.where(d < 1.0, env, 0.0)
    rbf = env * jnp.sin(d * freqs_ref[...])
    ew_ref[...] = jnp.maximum(
        jnp.dot(rbf, wrbf_ref[...], preferred_element_type=jnp.float32), 0.0)


def _ew_tc(d2, freqs_row, W_rbf):
    return pl.pallas_call(
        _ew_tc_body,
        grid=(N_EDGES // _BE,),
        in_specs=[
            pl.BlockSpec((_BE, 1), lambda i: (i, 0)),
            pl.BlockSpec((1, N_RBF), lambda i: (0, 0)),
            pl.BlockSpec((N_RBF, DIM), lambda i: (0, 0)),
        ],
        out_specs=pl.BlockSpec((_BE, DIM), lambda i: (i, 0)),
        out_shape=jax.ShapeDtypeStruct((N_EDGES, DIM), jnp.float32),
    )(d2, freqs_row, W_rbf)


def _upd_tc_body(x_ref, g_ref, wm_ref, wu_ref, xo_ref):
    gsum = g_ref[0] + g_ref[1]
    aggr = jnp.dot(gsum, wm_ref[...], preferred_element_type=jnp.float32)
    h = jnp.dot(aggr, wu_ref[...], preferred_element_type=jnp.float32)
    xo_ref[...] = jnp.maximum(x_ref[...] + h, 0.0)


def _upd_tc(x, g, Wm, Wu):
    return pl.pallas_call(
        _upd_tc_body,
        grid=(N_NODES // _BN,),
        in_specs=[
            pl.BlockSpec((_BN, DIM), lambda i: (i, 0)),
            pl.BlockSpec((NC, _BN, DIM), lambda i: (0, i, 0)),
            pl.BlockSpec((DIM, DIM), lambda i: (0, 0)),
            pl.BlockSpec((DIM, DIM), lambda i: (0, 0)),
        ],
        out_specs=pl.BlockSpec((_BN, DIM), lambda i: (i, 0)),
        out_shape=jax.ShapeDtypeStruct((N_NODES, DIM), jnp.float32),
    )(x, g, Wm, Wu)


def _final_tc_body(x_ref, g_ref, wm_ref, wup_ref, wo_ref, out_ref):
    gsum = g_ref[0] + g_ref[1]
    aggr = jnp.dot(gsum, wm_ref[...], preferred_element_type=jnp.float32)
    h = jnp.dot(aggr, wup_ref[...], preferred_element_type=jnp.float32)
    x2 = jnp.maximum(x_ref[...] + h, 0.0)
    out_ref[...] = jnp.dot(x2, wo_ref[...], preferred_element_type=jnp.float32)


def _final_tc(x, g, Wm, Wup, Wo_pad):
    return pl.pallas_call(
        _final_tc_body,
        grid=(N_NODES // _BN,),
        in_specs=[
            pl.BlockSpec((_BN, DIM), lambda i: (i, 0)),
            pl.BlockSpec((NC, _BN, DIM), lambda i: (0, i, 0)),
            pl.BlockSpec((DIM, DIM), lambda i: (0, 0)),
            pl.BlockSpec((DIM, DIM), lambda i: (0, 0)),
            pl.BlockSpec((DIM, DIM), lambda i: (0, 0)),
        ],
        out_specs=pl.BlockSpec((_BN, DIM), lambda i: (i, 0)),
        out_shape=jax.ShapeDtypeStruct((N_NODES, DIM), jnp.float32),
    )(x, g, Wm, Wup, Wo_pad)


@jax.jit
def _run(pos, edge_index, W_init, freqs, W_rbf, W_msg, W_upd, W_out):
    src = edge_index[0]
    dst = edge_index[1]
    px = pos[:, 0]
    py = pos[:, 1]
    pz = pos[:, 2]

    d2 = _dist2_call(px, py, pz, src, dst)
    ew = _ew_tc(d2.reshape(N_EDGES, 1), freqs.reshape(1, N_RBF), W_rbf)
    x = _init_tc(pos, W_init)

    g = _gather_mul_scatter(x, ew, src, dst)
    x = _upd_tc(x, g, W_msg[0], W_upd[0])

    g = _gather_mul_scatter(x, ew, src, dst)
    Wo_pad = jnp.pad(W_out, ((0, 0), (0, DIM - OUT_DIM)))
    out = _final_tc(x, g, W_msg[1], W_upd[1], Wo_pad)
    return out[:, :OUT_DIM]


def kernel(pos, edge_index, W_init, freqs, W_rbf, W_msg, W_upd, W_out):
    return _run(pos, edge_index, W_init, freqs, W_rbf, W_msg, W_upd, W_out)
